# baseline (speedup 1.0000x reference)
import functools

import jax
import jax.numpy as jnp
import numpy as np
from jax import lax
from jax.experimental import pallas as pl
from jax.experimental.pallas import tpu as pltpu

ROWS, LANES = 2048, 128
P = ROWS * LANES
LOG2P = 18
NSTAGE = LOG2P * (LOG2P + 1) // 2


def _stage_params():
    rows = []
    for kk in range(1, LOG2P + 1):
        K = 1 << kk
        for jj in range(kk - 1, -1, -1):
            J = 1 << jj
            if J >= LANES:
                r = J // LANES
                rows.append((K, J, 1, (ROWS - r) % ROWS, r))
            else:
                rows.append((K, J, 0, (LANES - J) % LANES, J))
    return np.asarray(rows, dtype=np.int32)


def _body(params_ref, logits_ref, target_ref, out_ref, x_ref, g_ref):
    b = pl.program_id(0)
    s = pl.program_id(1)

    row = lax.broadcasted_iota(jnp.int32, (ROWS, LANES), 0)
    lane = lax.broadcasted_iota(jnp.int32, (ROWS, LANES), 1)
    flat = row * LANES + lane

    @pl.when(s == 0)
    def _pack():
        lab = target_ref[0]
        labf = lab.astype(jnp.float32)
        e = 1.0 - logits_ref[0] * (2.0 * labf - 1.0)
        epos = jnp.maximum(e, 0.0)
        bits = (lax.bitcast_convert_type(epos, jnp.int32) & jnp.int32(~1)) | lab
        x_ref[...] = lax.bitcast_convert_type(bits, jnp.float32)
        g_ref[0] = jnp.sum(labf)

    K = params_ref[s, 0]
    J = params_ref[s, 1]
    is_row = params_ref[s, 2]
    sh_m = params_ref[s, 3]
    sh_p = params_ref[s, 4]

    x = x_ref[...]
    low = (flat & J) == 0
    desc = (flat & K) == 0

    xm, xp_ = lax.cond(
        is_row == 1,
        lambda: (pltpu.roll(x, sh_m, axis=0), pltpu.roll(x, sh_p, axis=0)),
        lambda: (pltpu.roll(x, sh_m, axis=1), pltpu.roll(x, sh_p, axis=1)),
    )
    partner = jnp.where(low, xm, xp_)
    keep_max = low == desc
    x_ref[...] = jnp.where(keep_max, jnp.maximum(x, partner),
                           jnp.minimum(x, partner))

    @pl.when(s == NSTAGE - 1)
    def _eval():
        sbits = lax.bitcast_convert_type(x_ref[...], jnp.int32)
        l_sorted = (sbits & 1).astype(jnp.float32)
        e_sorted = lax.bitcast_convert_type(sbits & jnp.int32(~1), jnp.float32)

        cs = l_sorted
        for sh in (1, 2, 4, 8, 16, 32, 64):
            cs = cs + jnp.where(lane >= sh, pltpu.roll(cs, sh, axis=1), 0.0)
        rt = cs[:, LANES - 1:LANES]
        rs = rt
        rowv = lax.broadcasted_iota(jnp.int32, (ROWS, 1), 0)
        for sh in (1, 2, 4, 8, 16, 32, 64, 128, 256, 512, 1024):
            rs = rs + jnp.where(rowv >= sh, pltpu.roll(rs, sh, axis=0), 0.0)
        c = cs + (rs - rt)

        G = g_ref[0]
        k = flat.astype(jnp.float32) + 1.0
        n = k - c
        cm1 = c - l_sorted
        nm1 = (k - 1.0) - cm1
        jk = k / (G + n)
        jm1 = (k - 1.0) / jnp.maximum(G + nm1, 1.0)
        loss = jnp.sum(e_sorted * (jk - jm1))

        prev = jnp.where(b == 0, 0.0, out_ref[0, 0])
        out_ref[0, 0] = prev + loss * (1.0 / 8.0)


@functools.partial(jax.jit, static_argnames=("interpret",))
def _run(logits, target, interpret=False):
    B = logits.shape[0]
    lg = logits.reshape(B, ROWS, LANES)
    tg = target.reshape(B, ROWS, LANES)
    params = jnp.asarray(_stage_params())

    out = pl.pallas_call(
        _body,
        grid=(B, NSTAGE),
        in_specs=[
            pl.BlockSpec(memory_space=pltpu.SMEM),
            pl.BlockSpec((1, ROWS, LANES), lambda b, s: (b, 0, 0)),
            pl.BlockSpec((1, ROWS, LANES), lambda b, s: (b, 0, 0)),
        ],
        out_specs=pl.BlockSpec((1, 1), lambda b, s: (0, 0),
                               memory_space=pltpu.SMEM),
        out_shape=jax.ShapeDtypeStruct((1, 1), jnp.float32),
        scratch_shapes=[
            pltpu.VMEM((ROWS, LANES), jnp.float32),
            pltpu.SMEM((1,), jnp.float32),
        ],
        interpret=interpret,
    )(params, lg, tg)
    return out.reshape(())


def kernel(logits, target):
    return _run(logits, target)


# speedup vs baseline: 1.3458x; 1.3458x over previous
import functools

import jax
import jax.numpy as jnp
import numpy as np
from jax import lax
from jax.experimental import pallas as pl
from jax.experimental.pallas import tpu as pltpu

ROWS, LANES = 2048, 128
P = ROWS * LANES
LOG2P = 18
NSTAGE = LOG2P * (LOG2P + 1) // 2


FUSE = 3


def _stage_params():
    rows = []
    for kk in range(1, LOG2P + 1):
        K = 1 << kk
        for jj in range(kk - 1, -1, -1):
            J = 1 << jj
            k_row = K >= LANES * 2 or kk == LOG2P
            kbit = (K // LANES) if k_row else K
            if J >= LANES:
                r = J // LANES
                rows.append((0, r, kbit, (ROWS - r) % ROWS, r))
            elif k_row:
                rows.append((1, J, kbit, (LANES - J) % LANES, J))
            else:
                rows.append((2, J, kbit, (LANES - J) % LANES, J))
    return np.asarray(rows, dtype=np.int32)


def _body(params_ref, logits_ref, target_ref, out_ref, x_ref, g_ref):
    b = pl.program_id(0)
    s = pl.program_id(1)

    row = lax.broadcasted_iota(jnp.int32, (ROWS, LANES), 0)
    lane = lax.broadcasted_iota(jnp.int32, (ROWS, LANES), 1)

    @pl.when(s == 0)
    def _pack():
        lab = target_ref[0]
        labf = lab.astype(jnp.float32)
        e = 1.0 - logits_ref[0] * (2.0 * labf - 1.0)
        epos = jnp.maximum(e, 0.0)
        bits = (lax.bitcast_convert_type(epos, jnp.int32) & jnp.int32(~1)) | lab
        x_ref[...] = lax.bitcast_convert_type(bits, jnp.float32)
        g_ref[0] = jnp.sum(labf)

    def _substage(t, x):
        i = s * FUSE + t
        case = params_ref[i, 0]
        jbit = params_ref[i, 1]
        kbit = params_ref[i, 2]
        sh_m = params_ref[i, 3]
        sh_p = params_ref[i, 4]

        def _cx(x, low, desc, axis):
            xm = pltpu.roll(x, sh_m, axis=axis)
            xp_ = pltpu.roll(x, sh_p, axis=axis)
            partner = jnp.where(low, xm, xp_)
            keep_max = low == desc
            return jnp.where(keep_max, jnp.maximum(x, partner),
                             jnp.minimum(x, partner))

        return lax.switch(case, [
            lambda x: _cx(x, (row & jbit) == 0, (row & kbit) == 0, 0),
            lambda x: _cx(x, (lane & jbit) == 0, (row & kbit) == 0, 1),
            lambda x: _cx(x, (lane & jbit) == 0, (lane & kbit) == 0, 1),
        ], x)

    x = x_ref[...]
    for t in range(FUSE):
        x = _substage(t, x)
    x_ref[...] = x

    @pl.when(s == NSTAGE // FUSE - 1)
    def _eval():
        sbits = lax.bitcast_convert_type(x_ref[...], jnp.int32)
        l_sorted = (sbits & 1).astype(jnp.float32)
        e_sorted = lax.bitcast_convert_type(sbits & jnp.int32(~1), jnp.float32)

        cs = l_sorted
        for sh in (1, 2, 4, 8, 16, 32, 64):
            cs = cs + jnp.where(lane >= sh, pltpu.roll(cs, sh, axis=1), 0.0)
        rt = cs[:, LANES - 1:LANES]
        rs = rt
        rowv = lax.broadcasted_iota(jnp.int32, (ROWS, 1), 0)
        for sh in (1, 2, 4, 8, 16, 32, 64, 128, 256, 512, 1024):
            rs = rs + jnp.where(rowv >= sh, pltpu.roll(rs, sh, axis=0), 0.0)
        c = cs + (rs - rt)

        G = g_ref[0]
        k = (row * LANES + lane).astype(jnp.float32) + 1.0
        n = k - c
        cm1 = c - l_sorted
        nm1 = (k - 1.0) - cm1
        jk = k / (G + n)
        jm1 = (k - 1.0) / jnp.maximum(G + nm1, 1.0)
        loss = jnp.sum(e_sorted * (jk - jm1))

        prev = jnp.where(b == 0, 0.0, out_ref[0, 0])
        out_ref[0, 0] = prev + loss * (1.0 / 8.0)


@functools.partial(jax.jit, static_argnames=("interpret",))
def _run(logits, target, interpret=False):
    B = logits.shape[0]
    lg = logits.reshape(B, ROWS, LANES)
    tg = target.reshape(B, ROWS, LANES)
    params = jnp.asarray(_stage_params())

    out = pl.pallas_call(
        _body,
        grid=(B, NSTAGE // FUSE),
        in_specs=[
            pl.BlockSpec(memory_space=pltpu.SMEM),
            pl.BlockSpec((1, ROWS, LANES), lambda b, s: (b, 0, 0)),
            pl.BlockSpec((1, ROWS, LANES), lambda b, s: (b, 0, 0)),
        ],
        out_specs=pl.BlockSpec((1, 1), lambda b, s: (0, 0),
                               memory_space=pltpu.SMEM),
        out_shape=jax.ShapeDtypeStruct((1, 1), jnp.float32),
        scratch_shapes=[
            pltpu.VMEM((ROWS, LANES), jnp.float32),
            pltpu.SMEM((1,), jnp.float32),
        ],
        interpret=interpret,
    )(params, lg, tg)
    return out.reshape(())


def kernel(logits, target):
    return _run(logits, target)
